# Initial kernel scaffold; baseline (speedup 1.0000x reference)
import functools

import jax
import jax.numpy as jnp
from jax import lax
from jax.experimental import pallas as pl
from jax.experimental.pallas import tpu as pltpu
from jax.experimental.pallas import tpu_sc as plsc

N = 10000
E = 160000
D_IN = 256
D_HID = 512
D_OUT = 256

NC = 2
NS = 16
NW = NC * NS
L = 16
EPW = E // NW
EPT = E // NS
NPAD = 10240
NEG = -1e30


def _iota():
    return lax.iota(jnp.int32, L)


def _make_score_kernel(C):
    CC = C // L
    G = (EPW + L - 1) // L

    mesh = plsc.VectorSubcoreMesh(core_axis_name="c", subcore_axis_name="s")

    @functools.partial(
        pl.kernel,
        out_type=(
            jax.ShapeDtypeStruct((E,), jnp.float32),
            jax.ShapeDtypeStruct((NW, L), jnp.float32),
        ),
        mesh=mesh,
        scratch_types=[
            pltpu.VMEM((G * L,), jnp.int32),
            pltpu.VMEM((G * L,), jnp.int32),
            pltpu.VMEM((G * L,), jnp.float32),
            pltpu.VMEM((C,), jnp.float32),
            pltpu.VMEM((L, C), jnp.float32),
            pltpu.VMEM((L, C), jnp.float32),
            pltpu.VMEM((L,), jnp.float32),
            pltpu.SemaphoreType.DMA,
            pltpu.SemaphoreType.DMA,
        ],
    )
    def score_kernel(xl_hbm, xr_hbm, src_hbm, dst_hbm, att_hbm,
                     s_hbm, tmax_hbm,
                     src_v, dst_v, s_v, att_v, gl_v, gr_v, tm_v, sem1, sem2):
        wid = lax.axis_index("c") * NS + lax.axis_index("s")
        base = wid * EPW
        pltpu.sync_copy(src_hbm.at[pl.ds(base, EPW)], src_v.at[pl.ds(0, EPW)])
        pltpu.sync_copy(dst_hbm.at[pl.ds(base, EPW)], dst_v.at[pl.ds(0, EPW)])
        pltpu.sync_copy(att_hbm, att_v)
        att_regs = [att_v[pl.ds(c * L, L)] for c in range(CC)]
        lanes = _iota()

        def group_body(g, tm):
            vmask = (g * L + lanes) < EPW
            sidx = jnp.where(vmask, src_v[pl.ds(g * L, L)], 0)
            didx = jnp.where(vmask, dst_v[pl.ds(g * L, L)], 0)
            cp1 = pltpu.async_copy(xl_hbm.at[sidx], gl_v, sem1)
            cp2 = pltpu.async_copy(xr_hbm.at[didx], gr_v, sem2)
            cp1.wait()
            cp2.wait()

            def edge_body(b, svec):
                acc = jnp.zeros((L,), jnp.float32)
                for c in range(CC):
                    z = gl_v[b, pl.ds(c * L, L)] + gr_v[b, pl.ds(c * L, L)]
                    lk = jnp.maximum(z, 0.2 * z)
                    acc = acc + lk * att_regs[c]
                sco = jnp.sum(acc)
                return jnp.where(lanes == b, sco, svec)

            svec = lax.fori_loop(0, L, edge_body, jnp.zeros((L,), jnp.float32))
            s_v[pl.ds(g * L, L)] = svec
            return jnp.maximum(tm, jnp.where(vmask, svec, NEG))

        tm = lax.fori_loop(0, G, group_body, jnp.full((L,), NEG, jnp.float32))
        tm_v[...] = tm
        pltpu.sync_copy(s_v.at[pl.ds(0, EPW)], s_hbm.at[pl.ds(base, EPW)])
        pltpu.sync_copy(tm_v, tmax_hbm.at[wid])

    return score_kernel


def _make_agg_kernel(C, slab):
    CC = C // L
    CP = C + L
    nslab = NPAD // (NC * slab)
    slab16 = slab // NS
    OB = 16
    CAP = EPT + L

    mesh = plsc.VectorSubcoreMesh(core_axis_name="c", subcore_axis_name="s")

    @functools.partial(
        pl.kernel,
        out_type=jax.ShapeDtypeStruct((NPAD, C), jnp.float32),
        mesh=mesh,
        scratch_types=[
            pltpu.VMEM((EPT,), jnp.int32),
            pltpu.VMEM((EPT,), jnp.int32),
            pltpu.VMEM((EPT,), jnp.float32),
            pltpu.VMEM((CAP,), jnp.int32),
            pltpu.VMEM((CAP,), jnp.int32),
            pltpu.VMEM((CAP,), jnp.float32),
            pltpu.VMEM((L, C), jnp.float32),
            pltpu.VMEM((L, CP), jnp.float32),
            pltpu.VMEM((OB, CP), jnp.float32),
            pltpu.VMEM((OB, C), jnp.float32),
            pltpu.VMEM((NW, L), jnp.float32),
            pltpu.VMEM_SHARED((slab, CP), jnp.float32),
            pltpu.SemaphoreType.DMA,
        ],
    )
    def agg_kernel(xl_hbm, src_hbm, dst_hbm, s_hbm, tmax_hbm, out_hbm,
                   src_v, dst_v, ex_v, csrc_v, cldst_v, cex_v,
                   grow_v, srow_v, ob_v, obo_v, tmx_v, acc_sh, sem):
        cid = lax.axis_index("c")
        sid = lax.axis_index("s")
        ebase = sid * EPT
        pltpu.sync_copy(src_hbm.at[pl.ds(ebase, EPT)], src_v)
        pltpu.sync_copy(dst_hbm.at[pl.ds(ebase, EPT)], dst_v)
        pltpu.sync_copy(s_hbm.at[pl.ds(ebase, EPT)], ex_v)
        pltpu.sync_copy(tmax_hbm, tmx_v)
        lanes = _iota()

        def max_body(r, m):
            return jnp.maximum(m, tmx_v[r, pl.ds(0, L)])

        kvec = lax.fori_loop(0, NW, max_body, jnp.full((L,), NEG, jnp.float32))
        kmax = jnp.max(kvec)

        def exp_body(g, carry):
            ex_v[pl.ds(g * L, L)] = jnp.exp(ex_v[pl.ds(g * L, L)] - kmax)
            return carry

        lax.fori_loop(0, EPT // L, exp_body, jnp.int32(0))

        for sl in range(nslab):
            slab_lo = (cid * nslab + sl) * slab

            def zrow_body(r, carry):
                for c in range(CP // L):
                    ob_v[r, pl.ds(c * L, L)] = jnp.zeros((L,), jnp.float32)
                return carry

            lax.fori_loop(0, OB, zrow_body, jnp.int32(0))

            def zcopy_body(k, carry):
                pltpu.sync_copy(
                    ob_v, acc_sh.at[pl.ds(sid * slab16 + k * OB, OB)])
                return carry

            lax.fori_loop(0, slab16 // OB, zcopy_body, jnp.int32(0))
            plsc.subcore_barrier()

            def scan_body(g, off):
                dv = dst_v[pl.ds(g * L, L)]
                ldst = dv - slab_lo
                mask = (ldst >= 0) & (ldst < slab)
                plsc.store_compressed(csrc_v.at[pl.ds(off, L)],
                                      src_v[pl.ds(g * L, L)], mask=mask)
                plsc.store_compressed(cldst_v.at[pl.ds(off, L)],
                                      jnp.where(mask, ldst, 0), mask=mask)
                plsc.store_compressed(cex_v.at[pl.ds(off, L)],
                                      ex_v[pl.ds(g * L, L)], mask=mask)
                return off + jnp.max(plsc.all_reduce_population_count(mask))

            off = lax.fori_loop(0, EPT // L, scan_body, jnp.int32(0))
            csrc_v[pl.ds(off, L)] = jnp.zeros((L,), jnp.int32)
            cldst_v[pl.ds(off, L)] = jnp.zeros((L,), jnp.int32)
            cex_v[pl.ds(off, L)] = jnp.zeros((L,), jnp.float32)
            nch = (off + L - 1) // L

            def chunk_body(j, carry):
                sidx = csrc_v[pl.ds(j * L, L)]
                pltpu.async_copy(xl_hbm.at[sidx], grow_v, sem).wait()

                def row_body(b, c2):
                    exs = plsc.load_gather(
                        cex_v, [jnp.broadcast_to(j * L + b, (L,))])
                    for c in range(CC):
                        srow_v[b, pl.ds(c * L, L)] = (
                            grow_v[b, pl.ds(c * L, L)] * exs)
                    srow_v[b, pl.ds(C, L)] = jnp.where(
                        lanes == 0, exs, jnp.zeros((L,), jnp.float32))
                    return c2

                lax.fori_loop(0, L, row_body, jnp.int32(0))
                ldidx = cldst_v[pl.ds(j * L, L)]
                pltpu.sync_copy(srow_v, acc_sh.at[ldidx], add=True)
                return carry

            lax.fori_loop(0, nch, chunk_body, jnp.int32(0))
            plsc.subcore_barrier()

            def out_body(k, carry):
                row0 = sid * slab16 + k * OB
                pltpu.sync_copy(acc_sh.at[pl.ds(row0, OB)], ob_v)

                def norm_body(r, c2):
                    den = plsc.load_gather(
                        ob_v, [jnp.broadcast_to(r, (L,)),
                               jnp.broadcast_to(jnp.int32(C), (L,))])
                    rec = 1.0 / (den + 1e-16)
                    for c in range(CC):
                        obo_v[r, pl.ds(c * L, L)] = (
                            ob_v[r, pl.ds(c * L, L)] * rec)
                    return c2

                lax.fori_loop(0, OB, norm_body, jnp.int32(0))
                pltpu.sync_copy(
                    obo_v, out_hbm.at[pl.ds(slab_lo + row0, OB)])
                return carry

            lax.fori_loop(0, slab16 // OB, out_body, jnp.int32(0))
            if sl + 1 < nslab:
                plsc.subcore_barrier()

    return agg_kernel


def _mm2(x, wl, wr, bias=None):
    n, k = x.shape
    c = wl.shape[1]
    bn = 1000

    def body(x_ref, wl_ref, wr_ref, b_ref, xl_ref, xr_ref):
        xb = x_ref[...]
        if bias is not None:
            xb = xb + b_ref[...]
            xb = jnp.where(xb > 0, xb, jnp.expm1(xb))
        xl_ref[...] = jnp.dot(xb, wl_ref[...],
                              preferred_element_type=jnp.float32)
        xr_ref[...] = jnp.dot(xb, wr_ref[...],
                              preferred_element_type=jnp.float32)

    b2d = jnp.zeros((1, k), jnp.float32) if bias is None else bias.reshape(1, k)
    return pl.pallas_call(
        body,
        grid=(n // bn,),
        in_specs=[
            pl.BlockSpec((bn, k), lambda i: (i, 0)),
            pl.BlockSpec((k, c), lambda i: (0, 0)),
            pl.BlockSpec((k, c), lambda i: (0, 0)),
            pl.BlockSpec((1, k), lambda i: (0, 0)),
        ],
        out_specs=[
            pl.BlockSpec((bn, c), lambda i: (i, 0)),
            pl.BlockSpec((bn, c), lambda i: (i, 0)),
        ],
        out_shape=[
            jax.ShapeDtypeStruct((n, c), jnp.float32),
            jax.ShapeDtypeStruct((n, c), jnp.float32),
        ],
    )(x, wl, wr, b2d)


def _log_softmax_bias(z, bias):
    n, c = z.shape
    bn = 1000

    def body(z_ref, b_ref, o_ref):
        t = z_ref[...] + b_ref[...]
        m = jnp.max(t, axis=1, keepdims=True)
        t = t - m
        lse = jnp.log(jnp.sum(jnp.exp(t), axis=1, keepdims=True))
        o_ref[...] = t - lse

    return pl.pallas_call(
        body,
        grid=(n // bn,),
        in_specs=[
            pl.BlockSpec((bn, c), lambda i: (i, 0)),
            pl.BlockSpec((1, c), lambda i: (0, 0)),
        ],
        out_specs=pl.BlockSpec((bn, c), lambda i: (i, 0)),
        out_shape=jax.ShapeDtypeStruct((n, c), jnp.float32),
    )(z, bias.reshape(1, c))


_score_hid = _make_score_kernel(D_HID)
_score_out = _make_score_kernel(D_OUT)
_agg_hid = _make_agg_kernel(D_HID, 2560)
_agg_out = _make_agg_kernel(D_OUT, 5120)


def kernel(x, edge_index, Wl1, Wr1, att1, b1, Wl2, Wr2, att2, b2):
    src = edge_index[0].astype(jnp.int32)
    dst = edge_index[1].astype(jnp.int32)

    xl1, xr1 = _mm2(x, Wl1, Wr1)
    s1, tm1 = _score_hid(xl1, xr1, src, dst, att1)
    z1 = _agg_hid(xl1, src, dst, s1, tm1)[:N]

    xl2, xr2 = _mm2(z1, Wl2, Wr2, bias=b1)
    s2, tm2 = _score_out(xl2, xr2, src, dst, att2)
    z2 = _agg_out(xl2, src, dst, s2, tm2)[:N]

    return _log_softmax_bias(z2, b2)


# speedup vs baseline: 2.2148x; 2.2148x over previous
import functools

import jax
import jax.numpy as jnp
from jax import lax
from jax.experimental import pallas as pl
from jax.experimental.pallas import tpu as pltpu
from jax.experimental.pallas import tpu_sc as plsc

N = 10000
E = 160000
D_IN = 256
D_HID = 512
D_OUT = 256

NC = 2
NS = 16
NW = NC * NS
L = 16
EPW = E // NW
EPT = E // NS
NPAD = 10240
NEG = -1e30


def _iota():
    return lax.iota(jnp.int32, L)


def _make_score_kernel(C):
    CC = C // L
    G = (EPW + L - 1) // L

    mesh = plsc.VectorSubcoreMesh(core_axis_name="c", subcore_axis_name="s")

    @functools.partial(
        pl.kernel,
        out_type=(
            jax.ShapeDtypeStruct((E,), jnp.float32),
            jax.ShapeDtypeStruct((NW, L), jnp.float32),
        ),
        mesh=mesh,
        scratch_types=[
            pltpu.VMEM((G * L,), jnp.int32),
            pltpu.VMEM((G * L,), jnp.int32),
            pltpu.VMEM((G * L,), jnp.float32),
            pltpu.VMEM((C,), jnp.float32),
            pltpu.VMEM((L, C), jnp.float32),
            pltpu.VMEM((L, C), jnp.float32),
            pltpu.VMEM((L,), jnp.float32),
            pltpu.SemaphoreType.DMA,
            pltpu.SemaphoreType.DMA,
        ],
        compiler_params=pltpu.CompilerParams(needs_layout_passes=False),
    )
    def score_kernel(xl_hbm, xr_hbm, src_hbm, dst_hbm, att_hbm,
                     s_hbm, tmax_hbm,
                     src_v, dst_v, s_v, att_v, gl_v, gr_v, tm_v, sem1, sem2):
        wid = lax.axis_index("c") * NS + lax.axis_index("s")
        base = wid * EPW
        pltpu.sync_copy(src_hbm.at[pl.ds(base, EPW)], src_v.at[pl.ds(0, EPW)])
        pltpu.sync_copy(dst_hbm.at[pl.ds(base, EPW)], dst_v.at[pl.ds(0, EPW)])
        pltpu.sync_copy(att_hbm, att_v)
        att_regs = [att_v[pl.ds(c * L, L)] for c in range(CC)]
        lanes = _iota()

        def group_body(g, tm):
            vmask = (g * L + lanes) < EPW
            sidx = jnp.where(vmask, src_v[pl.ds(g * L, L)], 0)
            didx = jnp.where(vmask, dst_v[pl.ds(g * L, L)], 0)
            cp1 = pltpu.async_copy(xl_hbm.at[sidx], gl_v, sem1)
            cp2 = pltpu.async_copy(xr_hbm.at[didx], gr_v, sem2)
            cp1.wait()
            cp2.wait()

            def edge_body(b, svec):
                acc = jnp.zeros((L,), jnp.float32)
                for c in range(CC):
                    z = gl_v[b, pl.ds(c * L, L)] + gr_v[b, pl.ds(c * L, L)]
                    lk = jnp.maximum(z, 0.2 * z)
                    acc = acc + lk * att_regs[c]
                sco = jnp.sum(acc)
                return jnp.where(lanes == b, sco, svec)

            svec = lax.fori_loop(0, L, edge_body, jnp.zeros((L,), jnp.float32))
            s_v[pl.ds(g * L, L)] = svec
            return jnp.maximum(tm, jnp.where(vmask, svec, NEG))

        tm = lax.fori_loop(0, G, group_body, jnp.full((L,), NEG, jnp.float32))
        tm_v[...] = tm
        pltpu.sync_copy(s_v.at[pl.ds(0, EPW)], s_hbm.at[pl.ds(base, EPW)])
        pltpu.sync_copy(tm_v, tmax_hbm.at[wid])

    return score_kernel


def _make_agg_kernel(C, slab):
    CC = C // L
    CPB = C // 128 + 1
    nslab = NPAD // (NC * slab)
    slab16 = slab // NS
    CB = 400
    NCH = EPT // CB
    CAP = CB + L

    mesh = plsc.VectorSubcoreMesh(core_axis_name="c", subcore_axis_name="s")

    @functools.partial(
        pl.kernel,
        out_type=jax.ShapeDtypeStruct((NPAD, C), jnp.float32),
        mesh=mesh,
        scratch_types=[
            pltpu.VMEM((CB,), jnp.int32),
            pltpu.VMEM((CB,), jnp.int32),
            pltpu.VMEM((CB,), jnp.float32),
            pltpu.VMEM((CAP,), jnp.int32),
            pltpu.VMEM((CAP,), jnp.int32),
            pltpu.VMEM((CAP,), jnp.float32),
            pltpu.VMEM((L, C), jnp.float32),
            pltpu.VMEM((CPB, L, 128), jnp.float32),
            pltpu.VMEM((NW, L), jnp.float32),
            pltpu.VMEM((1, L), jnp.int32),
        ] + [pltpu.VMEM_SHARED((slab, 128), jnp.float32)
             for _ in range(CPB)] + [
            pltpu.SemaphoreType.DMA,
        ],
        compiler_params=pltpu.CompilerParams(needs_layout_passes=False),
    )
    def agg_kernel(xl_hbm, src_hbm, dst_hbm, s_hbm, tmax_hbm, out_hbm,
                   src_v, dst_v, s_v, csrc_v, cldst_v, cex_v,
                   grow_v, srow_v, tmx_v, idx2_v, *accs_and_sem):
        accs = accs_and_sem[:CPB]
        sem = accs_and_sem[CPB]
        cid = lax.axis_index("c")
        sid = lax.axis_index("s")
        ebase = sid * EPT
        pltpu.sync_copy(tmax_hbm, tmx_v)
        lanes = _iota()

        def max_body(r, m):
            return jnp.maximum(m, tmx_v[r, pl.ds(0, L)])

        kvec = lax.fori_loop(0, NW, max_body, jnp.full((L,), NEG, jnp.float32))
        kmax = jnp.max(kvec)

        def flush(nch):
            def chunk_body(j, carry):
                sidx = csrc_v[pl.ds(j * L, L)]
                pltpu.async_copy(xl_hbm.at[sidx], grow_v, sem).wait()

                def row_body(b, c2):
                    exs = plsc.load_gather(
                        cex_v, [jnp.broadcast_to(j * L + b, (L,))])
                    for c in range(CC):
                        srow_v[c // 8, b, pl.ds((c % 8) * L, L)] = (
                            grow_v[b, pl.ds(c * L, L)] * exs)
                    srow_v[CPB - 1, b, pl.ds(0, L)] = jnp.where(
                        lanes == 0, exs, jnp.zeros((L,), jnp.float32))
                    return c2

                lax.fori_loop(0, L, row_body, jnp.int32(0))
                idx2_v[0, pl.ds(0, L)] = cldst_v[pl.ds(j * L, L)]
                for p in range(CPB):
                    pltpu.sync_copy(srow_v.at[p], accs[p].at[idx2_v.at[0]],
                                    add=True)
                return carry

            lax.fori_loop(0, nch, chunk_body, jnp.int32(0))

        for sl in range(nslab):
            slab_lo = (cid * nslab + sl) * slab

            def zrow_body(r, carry):
                for p in range(CPB):
                    for c in range(8):
                        srow_v[p, r, pl.ds(c * L, L)] = (
                            jnp.zeros((L,), jnp.float32))
                return carry

            lax.fori_loop(0, L, zrow_body, jnp.int32(0))

            def zcopy_body(k, carry):
                for p in range(CPB):
                    pltpu.sync_copy(
                        srow_v.at[0],
                        accs[p].at[pl.ds(sid * slab16 + k * L, L)])
                return carry

            lax.fori_loop(0, slab16 // L, zcopy_body, jnp.int32(0))
            plsc.subcore_barrier()

            def scan_chunk(ck, carry):
                cbase = ebase + ck * CB
                pltpu.sync_copy(src_hbm.at[pl.ds(cbase, CB)], src_v)
                pltpu.sync_copy(dst_hbm.at[pl.ds(cbase, CB)], dst_v)
                pltpu.sync_copy(s_hbm.at[pl.ds(cbase, CB)], s_v)

                def scan_body(g, off):
                    dv = dst_v[pl.ds(g * L, L)]
                    ldst = dv - slab_lo
                    mask = (ldst >= 0) & (ldst < slab)
                    ex = jnp.exp(s_v[pl.ds(g * L, L)] - kmax)
                    plsc.store_compressed(csrc_v.at[pl.ds(off, L)],
                                          src_v[pl.ds(g * L, L)], mask=mask)
                    plsc.store_compressed(cldst_v.at[pl.ds(off, L)],
                                          jnp.where(mask, ldst, 0), mask=mask)
                    plsc.store_compressed(cex_v.at[pl.ds(off, L)],
                                          ex, mask=mask)
                    cnt = plsc.all_reduce_population_count(mask)
                    return off + cnt[0]

                off = lax.fori_loop(0, CB // L, scan_body, jnp.int32(0))
                csrc_v[pl.ds(off, L)] = jnp.zeros((L,), jnp.int32)
                cldst_v[pl.ds(off, L)] = jnp.zeros((L,), jnp.int32)
                cex_v[pl.ds(off, L)] = jnp.zeros((L,), jnp.float32)
                flush((off + L - 1) // L)
                return carry

            lax.fori_loop(0, NCH, scan_chunk, jnp.int32(0))
            plsc.subcore_barrier()

            def out_body(k, carry):
                row0 = sid * slab16 + k * L
                for p in range(CPB):
                    pltpu.sync_copy(accs[p].at[pl.ds(row0, L)], srow_v.at[p])

                def norm_body(r, c2):
                    den = plsc.load_gather(
                        srow_v, [jnp.broadcast_to(jnp.int32(CPB - 1), (L,)),
                                 jnp.broadcast_to(r, (L,)),
                                 jnp.broadcast_to(jnp.int32(0), (L,))])
                    rec = 1.0 / (den + 1e-16)
                    for c in range(CC):
                        grow_v[r, pl.ds(c * L, L)] = (
                            srow_v[c // 8, r, pl.ds((c % 8) * L, L)] * rec)
                    return c2

                lax.fori_loop(0, L, norm_body, jnp.int32(0))
                pltpu.sync_copy(
                    grow_v, out_hbm.at[pl.ds(slab_lo + row0, L)])
                return carry

            lax.fori_loop(0, slab16 // L, out_body, jnp.int32(0))
            if sl + 1 < nslab:
                plsc.subcore_barrier()

    return agg_kernel


def _mm2(x, wl, wr, bias=None):
    n, k = x.shape
    c = wl.shape[1]
    bn = 1000

    def body(x_ref, wl_ref, wr_ref, b_ref, xl_ref, xr_ref):
        xb = x_ref[...]
        if bias is not None:
            xb = xb + b_ref[...]
            xb = jnp.where(xb > 0, xb, jnp.exp(jnp.minimum(xb, 0.0)) - 1.0)
        xl_ref[...] = jnp.dot(xb, wl_ref[...],
                              preferred_element_type=jnp.float32)
        xr_ref[...] = jnp.dot(xb, wr_ref[...],
                              preferred_element_type=jnp.float32)

    b2d = jnp.zeros((1, k), jnp.float32) if bias is None else bias.reshape(1, k)
    return pl.pallas_call(
        body,
        grid=(n // bn,),
        in_specs=[
            pl.BlockSpec((bn, k), lambda i: (i, 0)),
            pl.BlockSpec((k, c), lambda i: (0, 0)),
            pl.BlockSpec((k, c), lambda i: (0, 0)),
            pl.BlockSpec((1, k), lambda i: (0, 0)),
        ],
        out_specs=[
            pl.BlockSpec((bn, c), lambda i: (i, 0)),
            pl.BlockSpec((bn, c), lambda i: (i, 0)),
        ],
        out_shape=[
            jax.ShapeDtypeStruct((n, c), jnp.float32),
            jax.ShapeDtypeStruct((n, c), jnp.float32),
        ],
    )(x, wl, wr, b2d)


def _log_softmax_bias(z, bias):
    n, c = z.shape
    bn = 1000

    def body(z_ref, b_ref, o_ref):
        t = z_ref[...] + b_ref[...]
        m = jnp.max(t, axis=1, keepdims=True)
        t = t - m
        lse = jnp.log(jnp.sum(jnp.exp(t), axis=1, keepdims=True))
        o_ref[...] = t - lse

    return pl.pallas_call(
        body,
        grid=(n // bn,),
        in_specs=[
            pl.BlockSpec((bn, c), lambda i: (i, 0)),
            pl.BlockSpec((1, c), lambda i: (0, 0)),
        ],
        out_specs=pl.BlockSpec((bn, c), lambda i: (i, 0)),
        out_shape=jax.ShapeDtypeStruct((n, c), jnp.float32),
    )(z, bias.reshape(1, c))


_score_hid = _make_score_kernel(D_HID)
_score_out = _make_score_kernel(D_OUT)
_agg_hid = _make_agg_kernel(D_HID, 2560)
_agg_out = _make_agg_kernel(D_OUT, 2560)


def kernel(x, edge_index, Wl1, Wr1, att1, b1, Wl2, Wr2, att2, b2):
    src = edge_index[0].astype(jnp.int32)
    dst = edge_index[1].astype(jnp.int32)

    xl1, xr1 = _mm2(x, Wl1, Wr1)
    s1, tm1 = _score_hid(xl1, xr1, src, dst, att1)
    z1 = _agg_hid(xl1, src, dst, s1, tm1)[:N]

    xl2, xr2 = _mm2(z1, Wl2, Wr2, bias=b1)
    s2, tm2 = _score_out(xl2, xr2, src, dst, att2)
    z2 = _agg_out(xl2, src, dst, s2, tm2)[:N]

    return _log_softmax_bias(z2, b2)
